# baseline (speedup 1.0000x reference)
import functools
import math

import jax
import jax.numpy as jnp
from jax.experimental import pallas as pl
from jax.experimental.pallas import tpu as pltpu

E = 4
R = 16
D_IN = 2048
D_OUT = 2048
HID = 128
ALPHA = 32.0
SCALING = ALPHA / R
BALANCE_COEFF = 0.01

S_BLK = 2048


def _pass1_body(x_ref, a_ref, xa_ref, pooled_ref):
    s = pl.program_id(1)
    xb = x_ref[0]
    xa_ref[0] = jax.lax.dot_general(
        xb, a_ref[...], (((1,), (1,)), ((), ())),
        preferred_element_type=jnp.float32)
    psum = jnp.sum(xb, axis=0, keepdims=True)

    @pl.when(s == 0)
    def _init():
        pooled_ref[0] = psum

    @pl.when(s != 0)
    def _acc():
        pooled_ref[0] += psum


def _router_body(inv_seq, pooled_ref, rel_ref, reg_ref, w1x_ref, w1r_ref,
                 w1g_ref, b1_ref, w2_ref, b2_ref, erep_ref, gscale_ref,
                 bal_ref):
    pooled = pooled_ref[...] * inv_seq
    h = jax.lax.dot_general(pooled, w1x_ref[...], (((1,), (1,)), ((), ())),
                            preferred_element_type=jnp.float32)
    h += jax.lax.dot_general(rel_ref[...], w1r_ref[...], (((1,), (1,)), ((), ())),
                             preferred_element_type=jnp.float32)
    h += jax.lax.dot_general(reg_ref[...], w1g_ref[...], (((1,), (1,)), ((), ())),
                             preferred_element_type=jnp.float32)
    h += b1_ref[...]
    h = 0.5 * h * (1.0 + jax.lax.erf(h * (1.0 / math.sqrt(2.0))))
    logits = jax.lax.dot_general(h, w2_ref[...], (((1,), (1,)), ((), ())),
                                 preferred_element_type=jnp.float32)
    logits += b2_ref[...]
    m = jnp.max(logits, axis=-1, keepdims=True)
    p = jnp.exp(logits - m)
    p = p / jnp.sum(p, axis=-1, keepdims=True)
    gscale_ref[...] = jax.lax.dot_general(
        p, erep_ref[...], (((1,), (0,)), ((), ())),
        preferred_element_type=jnp.float32) * SCALING
    avg = jnp.mean(p, axis=0, keepdims=True)
    bal_ref[...] = BALANCE_COEFF * E * jnp.sum(avg * avg, axis=1,
                                               keepdims=True)


def _pass2_body(xa_ref, gscale_ref, b_ref, out_ref):
    xs = xa_ref[0] * gscale_ref[0]
    out_ref[0] = jax.lax.dot_general(
        xs, b_ref[...], (((1,), (0,)), ((), ())),
        preferred_element_type=jnp.float32)


def kernel(x, reliability_vec, regime_vec, lora_A, lora_B, W1, b1, W2, b2):
    B, S, d_in = x.shape
    e, r, _ = lora_A.shape
    er = e * r
    d_out = lora_B.shape[1]
    hid = W1.shape[0]

    a_mat = lora_A.reshape(er, d_in)
    b_mat = lora_B.transpose(0, 2, 1).reshape(er, d_out)
    w1x = W1[:, :d_in]
    w1r = W1[:, d_in:d_in + reliability_vec.shape[1]]
    w1g = W1[:, d_in + reliability_vec.shape[1]:]
    b1_2d = b1.reshape(1, hid)
    b2_2d = b2.reshape(1, e)
    erep = jnp.repeat(jnp.eye(e, dtype=jnp.float32), r, axis=1)

    ns = S // S_BLK
    xa, pooled = pl.pallas_call(
        _pass1_body,
        grid=(B, ns),
        in_specs=[
            pl.BlockSpec((1, S_BLK, d_in), lambda bb, ss: (bb, ss, 0)),
            pl.BlockSpec((er, d_in), lambda bb, ss: (0, 0)),
        ],
        out_specs=[
            pl.BlockSpec((1, S_BLK, er), lambda bb, ss: (bb, ss, 0)),
            pl.BlockSpec((1, 1, d_in), lambda bb, ss: (bb, 0, 0)),
        ],
        out_shape=[
            jax.ShapeDtypeStruct((B, S, er), jnp.float32),
            jax.ShapeDtypeStruct((B, 1, d_in), jnp.float32),
        ],
        compiler_params=pltpu.CompilerParams(
            dimension_semantics=("parallel", "arbitrary")),
    )(x, a_mat)
    pooled = pooled.reshape(B, d_in)

    gscale, bal = pl.pallas_call(
        functools.partial(_router_body, 1.0 / S),
        in_specs=[pl.BlockSpec(a.shape, lambda: (0,) * a.ndim) for a in (
            pooled, reliability_vec, regime_vec, w1x, w1r, w1g,
            b1_2d, W2, b2_2d, erep)],
        out_specs=[
            pl.BlockSpec((B, er), lambda: (0, 0)),
            pl.BlockSpec((1, 1), lambda: (0, 0)),
        ],
        out_shape=[
            jax.ShapeDtypeStruct((B, er), jnp.float32),
            jax.ShapeDtypeStruct((1, 1), jnp.float32),
        ],
    )(pooled, reliability_vec, regime_vec, w1x, w1r, w1g, b1_2d, W2, b2_2d,
      erep)

    out = pl.pallas_call(
        _pass2_body,
        grid=(B, ns),
        in_specs=[
            pl.BlockSpec((1, S_BLK, er), lambda bb, ss: (bb, ss, 0)),
            pl.BlockSpec((1, 1, er), lambda bb, ss: (bb, 0, 0)),
            pl.BlockSpec((er, d_out), lambda bb, ss: (0, 0)),
        ],
        out_specs=pl.BlockSpec((1, S_BLK, d_out), lambda bb, ss: (bb, ss, 0)),
        out_shape=jax.ShapeDtypeStruct((B, S, d_out), jnp.float32),
        compiler_params=pltpu.CompilerParams(
            dimension_semantics=("parallel", "parallel")),
    )(xa, gscale.reshape(B, 1, er), b_mat)

    return (out, bal.reshape(()))


# speedup vs baseline: 1.1188x; 1.1188x over previous
import functools
import math

import jax
import jax.numpy as jnp
from jax.experimental import pallas as pl
from jax.experimental.pallas import tpu as pltpu

E = 4
R = 16
ALPHA = 32.0
SCALING = ALPHA / R
BALANCE_COEFF = 0.01

S_BLK = 1024


def _pass1_body(inv_seq, x_ref, a_ref, rel_ref, reg_ref, w1x_ref, w1r_ref,
                w1g_ref, b1_ref, w2_ref, b2_ref, erep_ref,
                xa_ref, gscale_ref, bal_ref, pooled_ref):
    b = pl.program_id(0)
    s = pl.program_id(1)
    nb = pl.num_programs(0)
    ns = pl.num_programs(1)
    xb = x_ref[0]
    xa_ref[0] = jax.lax.dot_general(
        xb, a_ref[...], (((1,), (1,)), ((), ())),
        preferred_element_type=jnp.float32)
    psum = jnp.sum(xb, axis=0, keepdims=True)

    @pl.when(s == 0)
    def _init():
        pooled_ref[pl.ds(b, 1)] = psum

    @pl.when(s != 0)
    def _acc():
        pooled_ref[pl.ds(b, 1)] += psum

    @pl.when((b == nb - 1) & (s == ns - 1))
    def _router():
        pooled = pooled_ref[...] * inv_seq
        h = jax.lax.dot_general(pooled, w1x_ref[...], (((1,), (1,)), ((), ())),
                                preferred_element_type=jnp.float32)
        h += jax.lax.dot_general(rel_ref[...], w1r_ref[...],
                                 (((1,), (1,)), ((), ())),
                                 preferred_element_type=jnp.float32)
        h += jax.lax.dot_general(reg_ref[...], w1g_ref[...],
                                 (((1,), (1,)), ((), ())),
                                 preferred_element_type=jnp.float32)
        h += b1_ref[...]
        h = 0.5 * h * (1.0 + jax.lax.erf(h * (1.0 / math.sqrt(2.0))))
        logits = jax.lax.dot_general(h, w2_ref[...], (((1,), (1,)), ((), ())),
                                     preferred_element_type=jnp.float32)
        logits += b2_ref[...]
        m = jnp.max(logits, axis=-1, keepdims=True)
        p = jnp.exp(logits - m)
        p = p / jnp.sum(p, axis=-1, keepdims=True)
        gscale_ref[...] = jax.lax.dot_general(
            p, erep_ref[...], (((1,), (0,)), ((), ())),
            preferred_element_type=jnp.float32) * SCALING
        avg = jnp.mean(p, axis=0, keepdims=True)
        bal_ref[...] = BALANCE_COEFF * E * jnp.sum(avg * avg, axis=1,
                                                   keepdims=True)


def _pass2_body(xa_ref, gscale_ref, b_ref, out_ref):
    xs = xa_ref[0] * gscale_ref[0]
    out_ref[0] = jax.lax.dot_general(
        xs, b_ref[...], (((1,), (0,)), ((), ())),
        preferred_element_type=jnp.float32)


def kernel(x, reliability_vec, regime_vec, lora_A, lora_B, W1, b1, W2, b2):
    B, S, d_in = x.shape
    e, r, _ = lora_A.shape
    er = e * r
    d_out = lora_B.shape[1]
    hid = W1.shape[0]

    a_mat = lora_A.reshape(er, d_in)
    b_mat = lora_B.transpose(0, 2, 1).reshape(er, d_out)
    w1x = W1[:, :d_in]
    w1r = W1[:, d_in:d_in + reliability_vec.shape[1]]
    w1g = W1[:, d_in + reliability_vec.shape[1]:]
    b1_2d = b1.reshape(1, hid)
    b2_2d = b2.reshape(1, e)
    erep = jnp.repeat(jnp.eye(e, dtype=jnp.float32), r, axis=1)

    ns = S // S_BLK
    small = [reliability_vec, regime_vec, w1x, w1r, w1g, b1_2d, W2, b2_2d,
             erep]
    xa, gscale, bal = pl.pallas_call(
        functools.partial(_pass1_body, 1.0 / S),
        grid=(B, ns),
        in_specs=[pl.BlockSpec((1, S_BLK, d_in), lambda bb, ss: (bb, ss, 0)),
                  pl.BlockSpec((er, d_in), lambda bb, ss: (0, 0))]
                 + [pl.BlockSpec(a.shape, lambda bb, ss, n=None: (0, 0))
                    for a in small],
        out_specs=[
            pl.BlockSpec((1, S_BLK, er), lambda bb, ss: (bb, ss, 0)),
            pl.BlockSpec((B, er), lambda bb, ss: (0, 0)),
            pl.BlockSpec((1, 1), lambda bb, ss: (0, 0)),
        ],
        out_shape=[
            jax.ShapeDtypeStruct((B, S, er), jnp.float32),
            jax.ShapeDtypeStruct((B, er), jnp.float32),
            jax.ShapeDtypeStruct((1, 1), jnp.float32),
        ],
        scratch_shapes=[pltpu.VMEM((B, d_in), jnp.float32)],
        compiler_params=pltpu.CompilerParams(
            dimension_semantics=("arbitrary", "arbitrary")),
    )(x, a_mat, *small)

    out = pl.pallas_call(
        _pass2_body,
        grid=(B, ns),
        in_specs=[
            pl.BlockSpec((1, S_BLK, er), lambda bb, ss: (bb, ss, 0)),
            pl.BlockSpec((1, 1, er), lambda bb, ss: (bb, 0, 0)),
            pl.BlockSpec((er, d_out), lambda bb, ss: (0, 0)),
        ],
        out_specs=pl.BlockSpec((1, S_BLK, d_out), lambda bb, ss: (bb, ss, 0)),
        out_shape=jax.ShapeDtypeStruct((B, S, d_out), jnp.float32),
        compiler_params=pltpu.CompilerParams(
            dimension_semantics=("parallel", "parallel")),
    )(xa, gscale.reshape(B, 1, er), b_mat)

    return (out, bal.reshape(()))
